# baseline (speedup 1.0000x reference)
import functools

import jax
import jax.numpy as jnp
from jax import lax
from jax.experimental import pallas as pl
from jax.experimental.pallas import tpu as pltpu
from jax.experimental.pallas import tpu_sc as plsc

N = 2048
E = 65536
NSC = 2
NSUB = 16
CH = 256
W_CH = CH * N
PS = W_CH // NSUB
EPS = E // NSUB


def _sc_adj_body(e1_hbm, e2_hbm, e3_hbm, out_hbm,
                 src_v, dst_v, base_v, idx_v, zero_v, ones_v, acc, sem):
    c = lax.axis_index("c")
    s = lax.axis_index("s")

    def zinit(i, carry):
        zero_v[pl.ds(i * 16, 16)] = jnp.zeros((16,), jnp.float32)
        return carry

    lax.fori_loop(0, PS // 16, zinit, 0)

    def oinit(i, carry):
        ones_v[pl.ds(i * 16, 16)] = jnp.ones((16,), jnp.float32)
        return carry

    lax.fori_loop(0, EPS // 16, oinit, 0)
    dummy = W_CH + s * 128 + lax.iota(jnp.int32, 16) * 8

    for v, e_hbm in enumerate((e1_hbm, e2_hbm, e3_hbm)):
        cp1 = pltpu.async_copy(e_hbm.at[0, pl.ds(s * EPS, EPS)], src_v, sem)
        cp2 = pltpu.async_copy(e_hbm.at[1, pl.ds(s * EPS, EPS)], dst_v, sem)
        cp1.wait()
        cp2.wait()

        def bbody(i, carry):
            s16 = src_v[pl.ds(i * 16, 16)]
            d16 = dst_v[pl.ds(i * 16, 16)]
            base_v[pl.ds(i * 16, 16)] = s16 * N + d16
            return carry

        lax.fori_loop(0, EPS // 16, bbody, 0, unroll=4)

        for half in range(N // NSC // CH):
            r0 = c * (N // NSC) + half * CH
            lo = r0 * N
            pltpu.sync_copy(zero_v, acc.at[pl.ds(s * PS, PS)])
            plsc.subcore_barrier()

            def body(i, carry):
                b16 = base_v[pl.ds(i * 16, 16)]
                rel = b16 - lo
                inb = (rel >= 0) & (rel < W_CH)
                idx_v[pl.ds(i * 16, 16)] = jnp.where(inb, rel, dummy)
                return carry

            lax.fori_loop(0, EPS // 16, body, 0, unroll=4)
            pltpu.sync_copy(ones_v, acc.at[idx_v], add=True)
            plsc.subcore_barrier()
            dst_off = v * (N * N) + r0 * N + s * PS
            pltpu.sync_copy(acc.at[pl.ds(s * PS, PS)],
                            out_hbm.at[pl.ds(dst_off, PS)])


def _build_adjacencies(e1, e2, e3):
    mesh = plsc.VectorSubcoreMesh(core_axis_name="c", subcore_axis_name="s")
    k = functools.partial(
        pl.kernel,
        mesh=mesh,
        out_type=jax.ShapeDtypeStruct((3 * N * N,), jnp.float32),
        scratch_types=[
            pltpu.VMEM((EPS,), jnp.int32),
            pltpu.VMEM((EPS,), jnp.int32),
            pltpu.VMEM((EPS,), jnp.int32),
            pltpu.VMEM((EPS,), jnp.int32),
            pltpu.VMEM((PS,), jnp.float32),
            pltpu.VMEM((EPS,), jnp.float32),
            pltpu.VMEM_SHARED((W_CH + NSUB * 128,), jnp.float32),
            pltpu.SemaphoreType.DMA,
        ],
    )(_sc_adj_body)
    return k(e1, e2, e3).reshape(3, N, N)


def _hilo(x):
    hi = x.astype(jnp.bfloat16)
    lo = (x - hi.astype(jnp.float32)).astype(jnp.bfloat16)
    return hi, lo


def _dot3(x, yh, yl):
    xh, xl = _hilo(x)
    d = lambda a, b: jnp.dot(a, b, preferred_element_type=jnp.float32)
    return d(xh, yh) + d(xh, yl) + d(xl, yh)


def _k_split_body(w1_ref, w2_ref, w1h_ref, w1l_ref, w2h_ref, w2l_ref):
    h1, l1 = _hilo(w1_ref[...])
    w1h_ref[...] = h1
    w1l_ref[...] = l1
    h2, l2 = _hilo(w2_ref[...])
    w2h_ref[...] = h2
    w2l_ref[...] = l2


def _split_weights(w1, w2):
    mk = lambda w: jax.ShapeDtypeStruct(w.shape, jnp.bfloat16)
    return pl.pallas_call(
        _k_split_body,
        out_shape=[mk(w1), mk(w1), mk(w2), mk(w2)],
    )(w1, w2)

def _k_adjin_deg_body(a_ref, adjin_ref, abf_ref, rs_ref, cs_ref):
    i = pl.program_id(0)
    a = a_ref[...]
    adjin_ref[...] = a[0] + a[1] + a[2]
    abf_ref[...] = a.astype(jnp.bfloat16)
    rs_ref[...] = jnp.sum(a, axis=2)[:, None, :]

    @pl.when(i == 0)
    def _():
        cs_ref[...] = jnp.zeros_like(cs_ref)

    cs_ref[...] += jnp.sum(a, axis=1)[:, None, :]


def _adjin_and_degrees(A):
    return pl.pallas_call(
        _k_adjin_deg_body,
        grid=(16,),
        in_specs=[pl.BlockSpec((3, 128, N), lambda i: (0, i, 0))],
        out_specs=[
            pl.BlockSpec((128, N), lambda i: (i, 0)),
            pl.BlockSpec((3, 128, N), lambda i: (0, i, 0)),
            pl.BlockSpec((3, 1, 128), lambda i: (0, 0, i)),
            pl.BlockSpec((3, 1, N), lambda i: (0, 0, 0)),
        ],
        out_shape=[
            jax.ShapeDtypeStruct((N, N), jnp.float32),
            jax.ShapeDtypeStruct((3, N, N), jnp.bfloat16),
            jax.ShapeDtypeStruct((3, 1, N), jnp.float32),
            jax.ShapeDtypeStruct((3, 1, N), jnp.float32),
        ],
    )(A)


def _k_hs3_body(x1_ref, x2_ref, x3_ref, w_ref, deg_ref, hsh_ref, hsl_ref):
    for v, x_ref in enumerate((x1_ref, x2_ref, x3_ref)):
        scale = lax.rsqrt(jnp.maximum(deg_ref[v, 0], 1.0))
        h = jnp.dot(x_ref[...], w_ref[v],
                    preferred_element_type=jnp.float32) * scale[:, None]
        hh, hl = _hilo(h)
        hsh_ref[v] = hh
        hsl_ref[v] = hl


def _scaled_proj3(x1, x2, x3, w, deg):
    f = w.shape[-1]
    mk = jax.ShapeDtypeStruct((3, N, f), jnp.bfloat16)
    return pl.pallas_call(
        _k_hs3_body,
        out_shape=[mk, mk],
    )(x1, x2, x3, w, deg)


def _k_agg_body(a_ref, hsh_ref, hsl_ref, cs_ref, b_ref, f_ref, *, act):
    a = a_ref[0]
    dn = (((0,), (0,)), ((), ()))
    agg = (lax.dot_general(a, hsh_ref[0], dn,
                           preferred_element_type=jnp.float32)
           + lax.dot_general(a, hsl_ref[0], dn,
                             preferred_element_type=jnp.float32))
    scale = lax.rsqrt(jnp.maximum(cs_ref[0, 0], 1.0))
    r = agg * scale[:, None] + b_ref[0, 0][None, :]
    f_ref[0] = jnp.maximum(r, 0.0) if act else r


def _agg3(A, hsh, hsl, cs, b, act):
    f = hsh.shape[-1]
    return pl.pallas_call(
        functools.partial(_k_agg_body, act=act),
        grid=(3, 16),
        in_specs=[
            pl.BlockSpec((1, N, 128), lambda v, i: (v, 0, i)),
            pl.BlockSpec((1, N, f), lambda v, i: (v, 0, 0)),
            pl.BlockSpec((1, N, f), lambda v, i: (v, 0, 0)),
            pl.BlockSpec((1, 1, 128), lambda v, i: (v, 0, i)),
            pl.BlockSpec((1, 1, f), lambda v, i: (v, 0, 0)),
        ],
        out_specs=pl.BlockSpec((1, 128, f), lambda v, i: (v, i, 0)),
        out_shape=jax.ShapeDtypeStruct((3, N, f), jnp.float32),
    )(A, hsh, hsl, cs, b)


def _k_fuse_body(f_ref, wf_ref, z_ref):
    acc = jnp.dot(f_ref[0], wf_ref[0], preferred_element_type=jnp.float32)
    acc += jnp.dot(f_ref[1], wf_ref[1], preferred_element_type=jnp.float32)
    acc += jnp.dot(f_ref[2], wf_ref[2], preferred_element_type=jnp.float32)
    m = jnp.max(acc, axis=1, keepdims=True)
    e = jnp.exp(acc - m)
    sm = e / jnp.sum(e, axis=1, keepdims=True)
    z_ref[...] = jnp.maximum(sm, 0.0)


def _fuse(fb, wf):
    return pl.pallas_call(
        _k_fuse_body,
        out_shape=jax.ShapeDtypeStruct((N, 64), jnp.float32),
    )(fb, wf)


def _k_gfn_body(adjin_ref, w1h_ref, w1l_ref, bg1_ref, w2h_ref, w2l_ref,
                bg2_ref, adjr_ref):
    h1 = _dot3(adjin_ref[...], w1h_ref[...], w1l_ref[...]) + bg1_ref[0][None, :]
    h1 = jnp.maximum(h1, 0.0)
    h2 = _dot3(h1, w2h_ref[...], w2l_ref[...]) + bg2_ref[0][None, :]
    adjr_ref[...] = jnp.round(jnp.clip(h2, 0.0, 1.0) + 0.1)


def _gfn(adjin, wg1, bg1, wg2, bg2):
    w1h, w1l, w2h, w2l = _split_weights(wg1, wg2)
    return pl.pallas_call(
        _k_gfn_body,
        grid=(8,),
        in_specs=[
            pl.BlockSpec((256, N), lambda i: (i, 0)),
            pl.BlockSpec((N, N // 2), lambda i: (0, 0)),
            pl.BlockSpec((N, N // 2), lambda i: (0, 0)),
            pl.BlockSpec((1, N // 2), lambda i: (0, 0)),
            pl.BlockSpec((N // 2, N), lambda i: (0, 0)),
            pl.BlockSpec((N // 2, N), lambda i: (0, 0)),
            pl.BlockSpec((1, N), lambda i: (0, 0)),
        ],
        out_specs=pl.BlockSpec((256, N), lambda i: (i, 0)),
        out_shape=jax.ShapeDtypeStruct((N, N), jnp.float32),
    )(adjin, w1h, w1l, bg1, w2h, w2l, bg2)


def _k_abuild_body(ar_ref, art_ref, A_ref, s_ref):
    i = pl.program_id(0)
    j = pl.program_id(1)
    apn = ar_ref[...] + art_ref[...].T
    ri = lax.broadcasted_iota(jnp.int32, (128, 128), 0) + i * 128
    ci = lax.broadcasted_iota(jnp.int32, (128, 128), 1) + j * 128
    ablk = (apn != 0).astype(jnp.float32) + (ri == ci).astype(jnp.float32)
    A_ref[...] = ablk

    @pl.when(j == 0)
    def _():
        s_ref[...] = jnp.zeros_like(s_ref)

    s_ref[...] += jnp.sum(ablk, axis=1)[None, :]


def _build_consensus(adj_r):
    return pl.pallas_call(
        _k_abuild_body,
        grid=(16, 16),
        in_specs=[
            pl.BlockSpec((128, 128), lambda i, j: (i, j)),
            pl.BlockSpec((128, 128), lambda i, j: (j, i)),
        ],
        out_specs=[
            pl.BlockSpec((128, 128), lambda i, j: (i, j)),
            pl.BlockSpec((1, 128), lambda i, j: (0, i)),
        ],
        out_shape=[
            jax.ShapeDtypeStruct((N, N), jnp.float32),
            jax.ShapeDtypeStruct((1, N), jnp.float32),
        ],
    )(adj_r, adj_r)


def _k_proj_body(x_ref, w_ref, s_ref, u_ref):
    scale = lax.rsqrt(jnp.maximum(s_ref[0], 1.0))
    u_ref[...] = jnp.dot(x_ref[...], w_ref[...],
                         preferred_element_type=jnp.float32) * scale[:, None]


def _scaled_proj(x, w, s):
    f = w.shape[-1]
    return pl.pallas_call(
        _k_proj_body,
        out_shape=jax.ShapeDtypeStruct((N, f), jnp.float32),
    )(x, w, s)


def _k_dconv_body(a_ref, u_ref, s_ref, b_ref, o_ref, *, act):
    agg = jnp.dot(a_ref[...], u_ref[...], preferred_element_type=jnp.float32)
    scale = lax.rsqrt(jnp.maximum(s_ref[0], 1.0))
    r = agg * scale[:, None] + b_ref[0][None, :]
    o_ref[...] = jnp.maximum(r, 0.0) if act else r


def _dconv(A, u, s, b, act):
    f = u.shape[-1]
    return pl.pallas_call(
        functools.partial(_k_dconv_body, act=act),
        grid=(16,),
        in_specs=[
            pl.BlockSpec((128, N), lambda i: (i, 0)),
            pl.BlockSpec((N, f), lambda i: (0, 0)),
            pl.BlockSpec((1, 128), lambda i: (0, i)),
            pl.BlockSpec((1, f), lambda i: (0, 0)),
        ],
        out_specs=pl.BlockSpec((128, f), lambda i: (i, 0)),
        out_shape=jax.ShapeDtypeStruct((N, f), jnp.float32),
    )(A, u, s, b)


def _k_rec_body(fmb_ref, fm_ref, rec_ref):
    rec_ref[...] = lax.dot_general(fmb_ref[...], fm_ref[...],
                                   (((1,), (1,)), ((), ())),
                                   preferred_element_type=jnp.float32)


def _rec(fm):
    return pl.pallas_call(
        _k_rec_body,
        grid=(16,),
        in_specs=[
            pl.BlockSpec((128, 32), lambda i: (i, 0)),
            pl.BlockSpec((N, 32), lambda i: (0, 0)),
        ],
        out_specs=pl.BlockSpec((128, N), lambda i: (i, 0)),
        out_shape=jax.ShapeDtypeStruct((N, N), jnp.float32),
    )(fm, fm)


def kernel(data1, data2, data3, edge_index1, edge_index2, edge_index3,
           W1a, b1a, W1b, b1b, W2a, b2a, W2b, b2b, W3a, b3a, W3b, b3b,
           Wf1, Wf2, Wf3, Wm1, bm1, Wm2, bm2, Wg1, bg1, Wg2, bg2):
    A = _build_adjacencies(edge_index1, edge_index2, edge_index3)

    adjin, Abf, rs, cs = _adjin_and_degrees(A)

    wa = jnp.stack([W1a, W2a, W3a])
    ba = jnp.stack([b1a, b2a, b3a])[:, None, :]
    wb = jnp.stack([W1b, W2b, W3b])
    bb = jnp.stack([b1b, b2b, b3b])[:, None, :]
    wf = jnp.stack([Wf1, Wf2, Wf3])

    hsh_a, hsl_a = _scaled_proj3(data1, data2, data3, wa, rs)
    f_a = _agg3(Abf, hsh_a, hsl_a, cs, ba, act=True)
    hsh_b, hsl_b = _scaled_proj3(f_a[0], f_a[1], f_a[2], wb, rs)
    f_b = _agg3(Abf, hsh_b, hsl_b, cs, bb, act=False)

    z = _fuse(f_b, wf)

    adj_r = _gfn(adjin, Wg1, bg1[None, :], Wg2, bg2[None, :])
    Acons, s = _build_consensus(adj_r)

    u1 = _scaled_proj(z, Wm1, s)
    fm1 = _dconv(Acons, u1, s, bm1[None, :], act=True)
    u2 = _scaled_proj(fm1, Wm2, s)
    fm = _dconv(Acons, u2, s, bm2[None, :], act=False)

    rec = _rec(fm)
    return (adj_r, rec, rec, rec, fm)


# speedup vs baseline: 1.6064x; 1.6064x over previous
import functools

import jax
import jax.numpy as jnp
from jax import lax
from jax.experimental import pallas as pl
from jax.experimental.pallas import tpu as pltpu
from jax.experimental.pallas import tpu_sc as plsc

N = 2048
E = 65536
NSC = 2
NSUB = 16
CH = 256
W_CH = CH * N
PS = W_CH // NSUB
EPS = E // NSUB


def _sc_adj_body(e1_hbm, e2_hbm, e3_hbm, out_hbm,
                 src_v, dst_v, base_v, idx_v, zero_v, ones_v, acc, sem):
    c = lax.axis_index("c")
    s = lax.axis_index("s")

    def zinit(i, carry):
        zero_v[pl.ds(i * 16, 16)] = jnp.zeros((16,), jnp.float32)
        return carry

    lax.fori_loop(0, PS // 16, zinit, 0)

    def oinit(i, carry):
        ones_v[pl.ds(i * 16, 16)] = jnp.ones((16,), jnp.float32)
        return carry

    lax.fori_loop(0, EPS // 16, oinit, 0)
    dummy = W_CH + s * 128 + lax.iota(jnp.int32, 16) * 8

    for v, e_hbm in enumerate((e1_hbm, e2_hbm, e3_hbm)):
        cp1 = pltpu.async_copy(e_hbm.at[0, pl.ds(s * EPS, EPS)], src_v, sem)
        cp2 = pltpu.async_copy(e_hbm.at[1, pl.ds(s * EPS, EPS)], dst_v, sem)
        cp1.wait()
        cp2.wait()

        def bbody(i, carry):
            s16 = src_v[pl.ds(i * 16, 16)]
            d16 = dst_v[pl.ds(i * 16, 16)]
            base_v[pl.ds(i * 16, 16)] = s16 * N + d16
            return carry

        lax.fori_loop(0, EPS // 16, bbody, 0, unroll=4)

        for half in range(N // NSC // CH):
            r0 = c * (N // NSC) + half * CH
            lo = r0 * N
            pltpu.sync_copy(zero_v, acc.at[pl.ds(s * PS, PS)])
            plsc.subcore_barrier()

            def body(i, carry):
                b16 = base_v[pl.ds(i * 16, 16)]
                rel = b16 - lo
                inb = (rel >= 0) & (rel < W_CH)
                idx_v[pl.ds(i * 16, 16)] = jnp.where(inb, rel, dummy)
                return carry

            lax.fori_loop(0, EPS // 16, body, 0, unroll=4)
            pltpu.sync_copy(ones_v, acc.at[idx_v], add=True)
            plsc.subcore_barrier()
            dst_off = v * (N * N) + r0 * N + s * PS
            pltpu.sync_copy(acc.at[pl.ds(s * PS, PS)],
                            out_hbm.at[pl.ds(dst_off, PS)])


def _build_adjacencies(e1, e2, e3):
    mesh = plsc.VectorSubcoreMesh(core_axis_name="c", subcore_axis_name="s")
    k = functools.partial(
        pl.kernel,
        mesh=mesh,
        out_type=jax.ShapeDtypeStruct((3 * N * N,), jnp.float32),
        scratch_types=[
            pltpu.VMEM((EPS,), jnp.int32),
            pltpu.VMEM((EPS,), jnp.int32),
            pltpu.VMEM((EPS,), jnp.int32),
            pltpu.VMEM((EPS,), jnp.int32),
            pltpu.VMEM((PS,), jnp.float32),
            pltpu.VMEM((EPS,), jnp.float32),
            pltpu.VMEM_SHARED((W_CH + NSUB * 128,), jnp.float32),
            pltpu.SemaphoreType.DMA,
        ],
    )(_sc_adj_body)
    return k(e1, e2, e3).reshape(3, N, N)


def _hilo(x):
    hi = x.astype(jnp.bfloat16)
    lo = (x - hi.astype(jnp.float32)).astype(jnp.bfloat16)
    return hi, lo


def _k_adjin_deg_body(a_ref, adjin_ref, abf_ref, rs_ref, cs_ref):
    i = pl.program_id(0)
    a = a_ref[...]
    adjin_ref[...] = a[0] + a[1] + a[2]
    abf_ref[...] = a.astype(jnp.bfloat16)
    rs_ref[...] = jnp.sum(a, axis=2)[:, None, :]

    @pl.when(i == 0)
    def _():
        cs_ref[...] = jnp.zeros_like(cs_ref)

    cs_ref[...] += jnp.sum(a, axis=1)[:, None, :]


def _adjin_and_degrees(A):
    return pl.pallas_call(
        _k_adjin_deg_body,
        grid=(16,),
        in_specs=[pl.BlockSpec((3, 128, N), lambda i: (0, i, 0))],
        out_specs=[
            pl.BlockSpec((128, N), lambda i: (i, 0)),
            pl.BlockSpec((3, 128, N), lambda i: (0, i, 0)),
            pl.BlockSpec((3, 1, 128), lambda i: (0, 0, i)),
            pl.BlockSpec((3, 1, N), lambda i: (0, 0, 0)),
        ],
        out_shape=[
            jax.ShapeDtypeStruct((N, N), jnp.float32),
            jax.ShapeDtypeStruct((3, N, N), jnp.bfloat16),
            jax.ShapeDtypeStruct((3, 1, N), jnp.float32),
            jax.ShapeDtypeStruct((3, 1, N), jnp.float32),
        ],
    )(A)


def _k_hs3_body(x1_ref, x2_ref, x3_ref, w_ref, deg_ref, hsh_ref, hsl_ref):
    for v, x_ref in enumerate((x1_ref, x2_ref, x3_ref)):
        scale = lax.rsqrt(jnp.maximum(deg_ref[v, 0], 1.0))
        h = jnp.dot(x_ref[...], w_ref[v],
                    preferred_element_type=jnp.float32) * scale[:, None]
        hh, hl = _hilo(h)
        hsh_ref[v] = hh
        hsl_ref[v] = hl


def _scaled_proj3(x1, x2, x3, w, deg):
    f = w.shape[-1]
    mk = jax.ShapeDtypeStruct((3, N, f), jnp.bfloat16)
    return pl.pallas_call(
        _k_hs3_body,
        out_shape=[mk, mk],
    )(x1, x2, x3, w, deg)


def _k_agg_body(a_ref, hsh_ref, hsl_ref, cs_ref, b_ref, f_ref, *, act):
    a = a_ref[0]
    dn = (((0,), (0,)), ((), ()))
    agg = (lax.dot_general(a, hsh_ref[0], dn,
                           preferred_element_type=jnp.float32)
           + lax.dot_general(a, hsl_ref[0], dn,
                             preferred_element_type=jnp.float32))
    scale = lax.rsqrt(jnp.maximum(cs_ref[0, 0], 1.0))
    r = agg * scale[:, None] + b_ref[0, 0][None, :]
    f_ref[0] = jnp.maximum(r, 0.0) if act else r


def _agg3(A, hsh, hsl, cs, b, act):
    f = hsh.shape[-1]
    return pl.pallas_call(
        functools.partial(_k_agg_body, act=act),
        grid=(3,),
        in_specs=[
            pl.BlockSpec((1, N, N), lambda v: (v, 0, 0)),
            pl.BlockSpec((1, N, f), lambda v: (v, 0, 0)),
            pl.BlockSpec((1, N, f), lambda v: (v, 0, 0)),
            pl.BlockSpec((1, 1, N), lambda v: (v, 0, 0)),
            pl.BlockSpec((1, 1, f), lambda v: (v, 0, 0)),
        ],
        out_specs=pl.BlockSpec((1, N, f), lambda v: (v, 0, 0)),
        out_shape=jax.ShapeDtypeStruct((3, N, f), jnp.float32),
    )(A, hsh, hsl, cs, b)


def _k_fuse_body(f_ref, wf_ref, z_ref):
    acc = jnp.dot(f_ref[0], wf_ref[0], preferred_element_type=jnp.float32)
    acc += jnp.dot(f_ref[1], wf_ref[1], preferred_element_type=jnp.float32)
    acc += jnp.dot(f_ref[2], wf_ref[2], preferred_element_type=jnp.float32)
    m = jnp.max(acc, axis=1, keepdims=True)
    e = jnp.exp(acc - m)
    sm = e / jnp.sum(e, axis=1, keepdims=True)
    z_ref[...] = jnp.maximum(sm, 0.0)


def _fuse(fb, wf):
    return pl.pallas_call(
        _k_fuse_body,
        out_shape=jax.ShapeDtypeStruct((N, 64), jnp.float32),
    )(fb, wf)


def _k_gfn_body(adjin_ref, wg1_ref, bg1_ref, wg2_ref, bg2_ref, adjr_ref):
    h1 = jnp.dot(adjin_ref[...], wg1_ref[...],
                 preferred_element_type=jnp.float32) + bg1_ref[0][None, :]
    h1 = jnp.maximum(h1, 0.0)
    h2 = jnp.dot(h1, wg2_ref[...],
                 preferred_element_type=jnp.float32) + bg2_ref[0][None, :]
    adjr_ref[...] = jnp.round(jnp.clip(h2, 0.0, 1.0) + 0.1)


def _gfn(adjin, wg1, bg1, wg2, bg2):
    return pl.pallas_call(
        _k_gfn_body,
        grid=(8,),
        in_specs=[
            pl.BlockSpec((256, N), lambda i: (i, 0)),
            pl.BlockSpec((N, N // 2), lambda i: (0, 0)),
            pl.BlockSpec((1, N // 2), lambda i: (0, 0)),
            pl.BlockSpec((N // 2, N), lambda i: (0, 0)),
            pl.BlockSpec((1, N), lambda i: (0, 0)),
        ],
        out_specs=pl.BlockSpec((256, N), lambda i: (i, 0)),
        out_shape=jax.ShapeDtypeStruct((N, N), jnp.float32),
    )(adjin, wg1, bg1, wg2, bg2)


_CB = 512


def _k_abuild_body(ar_ref, art_ref, A_ref, s_ref):
    i = pl.program_id(0)
    j = pl.program_id(1)
    apn = ar_ref[...] + art_ref[...].T
    ri = lax.broadcasted_iota(jnp.int32, (_CB, _CB), 0) + i * _CB
    ci = lax.broadcasted_iota(jnp.int32, (_CB, _CB), 1) + j * _CB
    ablk = (apn != 0).astype(jnp.float32) + (ri == ci).astype(jnp.float32)
    A_ref[...] = ablk

    @pl.when(j == 0)
    def _():
        s_ref[...] = jnp.zeros_like(s_ref)

    s_ref[...] += jnp.sum(ablk, axis=1)[None, :]


def _build_consensus(adj_r):
    g = N // _CB
    return pl.pallas_call(
        _k_abuild_body,
        grid=(g, g),
        in_specs=[
            pl.BlockSpec((_CB, _CB), lambda i, j: (i, j)),
            pl.BlockSpec((_CB, _CB), lambda i, j: (j, i)),
        ],
        out_specs=[
            pl.BlockSpec((_CB, _CB), lambda i, j: (i, j)),
            pl.BlockSpec((1, _CB), lambda i, j: (0, i)),
        ],
        out_shape=[
            jax.ShapeDtypeStruct((N, N), jnp.float32),
            jax.ShapeDtypeStruct((1, N), jnp.float32),
        ],
    )(adj_r, adj_r)


def _k_proj_body(x_ref, w_ref, s_ref, u_ref):
    scale = lax.rsqrt(jnp.maximum(s_ref[0], 1.0))
    u_ref[...] = jnp.dot(x_ref[...], w_ref[...],
                         preferred_element_type=jnp.float32) * scale[:, None]


def _scaled_proj(x, w, s):
    f = w.shape[-1]
    return pl.pallas_call(
        _k_proj_body,
        out_shape=jax.ShapeDtypeStruct((N, f), jnp.float32),
    )(x, w, s)


def _k_dconv_body(a_ref, u_ref, s_ref, b_ref, o_ref, *, act):
    agg = jnp.dot(a_ref[...], u_ref[...], preferred_element_type=jnp.float32)
    scale = lax.rsqrt(jnp.maximum(s_ref[0], 1.0))
    r = agg * scale[:, None] + b_ref[0][None, :]
    o_ref[...] = jnp.maximum(r, 0.0) if act else r


def _dconv(A, u, s, b, act):
    f = u.shape[-1]
    return pl.pallas_call(
        functools.partial(_k_dconv_body, act=act),
        grid=(16,),
        in_specs=[
            pl.BlockSpec((128, N), lambda i: (i, 0)),
            pl.BlockSpec((N, f), lambda i: (0, 0)),
            pl.BlockSpec((1, 128), lambda i: (0, i)),
            pl.BlockSpec((1, f), lambda i: (0, 0)),
        ],
        out_specs=pl.BlockSpec((128, f), lambda i: (i, 0)),
        out_shape=jax.ShapeDtypeStruct((N, f), jnp.float32),
    )(A, u, s, b)


def _k_rec_body(fmb_ref, fm_ref, rec_ref):
    rec_ref[...] = lax.dot_general(fmb_ref[...], fm_ref[...],
                                   (((1,), (1,)), ((), ())),
                                   preferred_element_type=jnp.float32)


def _rec(fm):
    return pl.pallas_call(
        _k_rec_body,
        grid=(16,),
        in_specs=[
            pl.BlockSpec((128, 32), lambda i: (i, 0)),
            pl.BlockSpec((N, 32), lambda i: (0, 0)),
        ],
        out_specs=pl.BlockSpec((128, N), lambda i: (i, 0)),
        out_shape=jax.ShapeDtypeStruct((N, N), jnp.float32),
    )(fm, fm)


def kernel(data1, data2, data3, edge_index1, edge_index2, edge_index3,
           W1a, b1a, W1b, b1b, W2a, b2a, W2b, b2b, W3a, b3a, W3b, b3b,
           Wf1, Wf2, Wf3, Wm1, bm1, Wm2, bm2, Wg1, bg1, Wg2, bg2):
    A = _build_adjacencies(edge_index1, edge_index2, edge_index3)

    adjin, Abf, rs, cs = _adjin_and_degrees(A)

    wa = jnp.stack([W1a, W2a, W3a])
    ba = jnp.stack([b1a, b2a, b3a])[:, None, :]
    wb = jnp.stack([W1b, W2b, W3b])
    bb = jnp.stack([b1b, b2b, b3b])[:, None, :]
    wf = jnp.stack([Wf1, Wf2, Wf3])

    hsh_a, hsl_a = _scaled_proj3(data1, data2, data3, wa, rs)
    f_a = _agg3(Abf, hsh_a, hsl_a, cs, ba, act=True)
    hsh_b, hsl_b = _scaled_proj3(f_a[0], f_a[1], f_a[2], wb, rs)
    f_b = _agg3(Abf, hsh_b, hsl_b, cs, bb, act=False)

    z = _fuse(f_b, wf)

    adj_r = _gfn(adjin, Wg1, bg1[None, :], Wg2, bg2[None, :])
    Acons, s = _build_consensus(adj_r)

    u1 = _scaled_proj(z, Wm1, s)
    fm1 = _dconv(Acons, u1, s, bm1[None, :], act=True)
    u2 = _scaled_proj(fm1, Wm2, s)
    fm = _dconv(Acons, u2, s, bm2[None, :], act=False)

    rec = _rec(fm)
    return (adj_r, rec, rec, rec, fm)


# speedup vs baseline: 1.6090x; 1.0016x over previous
import functools

import jax
import jax.numpy as jnp
from jax import lax
from jax.experimental import pallas as pl
from jax.experimental.pallas import tpu as pltpu
from jax.experimental.pallas import tpu_sc as plsc

N = 2048
E = 65536
NSC = 2
NSUB = 16
CH = 256
W_CH = CH * N
PS = W_CH // NSUB
EPS = E // NSUB


def _sc_adj_body(e_hbm, out_hbm,
                 src_v, dst_v, base_v, idx_v, zero_v, ones_v, acc, sem):
    c = lax.axis_index("c")
    s = lax.axis_index("s")

    def zinit(i, carry):
        zero_v[pl.ds(i * 16, 16)] = jnp.zeros((16,), jnp.float32)
        return carry

    lax.fori_loop(0, PS // 16, zinit, 0)

    def oinit(i, carry):
        ones_v[pl.ds(i * 16, 16)] = jnp.ones((16,), jnp.float32)
        return carry

    lax.fori_loop(0, EPS // 16, oinit, 0)
    dummy = W_CH + s * 128 + lax.iota(jnp.int32, 16) * 8

    cp1 = pltpu.async_copy(e_hbm.at[0, pl.ds(s * EPS, EPS)], src_v, sem)
    cp2 = pltpu.async_copy(e_hbm.at[1, pl.ds(s * EPS, EPS)], dst_v, sem)
    cp1.wait()
    cp2.wait()

    def bbody(i, carry):
        s16 = src_v[pl.ds(i * 16, 16)]
        d16 = dst_v[pl.ds(i * 16, 16)]
        base_v[pl.ds(i * 16, 16)] = s16 * N + d16
        return carry

    lax.fori_loop(0, EPS // 16, bbody, 0, unroll=4)

    for half in range(N // NSC // CH):
        r0 = c * (N // NSC) + half * CH
        lo = r0 * N
        pltpu.sync_copy(zero_v, acc.at[pl.ds(s * PS, PS)])
        plsc.subcore_barrier()

        def body(i, carry):
            b16 = base_v[pl.ds(i * 16, 16)]
            rel = b16 - lo
            inb = (rel >= 0) & (rel < W_CH)
            idx_v[pl.ds(i * 16, 16)] = jnp.where(inb, rel, dummy)
            return carry

        lax.fori_loop(0, EPS // 16, body, 0, unroll=4)
        pltpu.sync_copy(ones_v, acc.at[idx_v], add=True)
        plsc.subcore_barrier()
        dst_off = r0 * N + s * PS
        pltpu.sync_copy(acc.at[pl.ds(s * PS, PS)],
                        out_hbm.at[pl.ds(dst_off, PS)])


def _build_adjacency(e):
    mesh = plsc.VectorSubcoreMesh(core_axis_name="c", subcore_axis_name="s")
    k = functools.partial(
        pl.kernel,
        mesh=mesh,
        out_type=jax.ShapeDtypeStruct((N * N,), jnp.float32),
        scratch_types=[
            pltpu.VMEM((EPS,), jnp.int32),
            pltpu.VMEM((EPS,), jnp.int32),
            pltpu.VMEM((EPS,), jnp.int32),
            pltpu.VMEM((EPS,), jnp.int32),
            pltpu.VMEM((PS,), jnp.float32),
            pltpu.VMEM((EPS,), jnp.float32),
            pltpu.VMEM_SHARED((W_CH + NSUB * 128,), jnp.float32),
            pltpu.SemaphoreType.DMA,
        ],
    )(_sc_adj_body)
    return k(e).reshape(N, N)


def _hilo(x):
    hi = x.astype(jnp.bfloat16)
    lo = (x - hi.astype(jnp.float32)).astype(jnp.bfloat16)
    return hi, lo


def _k_adjin_deg_body(a1_ref, a2_ref, a3_ref, adjin_ref, abf_ref,
                      rs_ref, cs_ref):
    i = pl.program_id(0)
    a1 = a1_ref[...]
    a2 = a2_ref[...]
    a3 = a3_ref[...]
    adjin_ref[...] = a1 + a2 + a3
    a = jnp.stack([a1, a2, a3])
    abf_ref[...] = a.astype(jnp.bfloat16)
    rs_ref[...] = jnp.sum(a, axis=2)[:, None, :]

    @pl.when(i == 0)
    def _():
        cs_ref[...] = jnp.zeros_like(cs_ref)

    cs_ref[...] += jnp.sum(a, axis=1)[:, None, :]


def _adjin_and_degrees(A1, A2, A3):
    blk = pl.BlockSpec((128, N), lambda i: (i, 0))
    return pl.pallas_call(
        _k_adjin_deg_body,
        grid=(16,),
        in_specs=[blk, blk, blk],
        out_specs=[
            pl.BlockSpec((128, N), lambda i: (i, 0)),
            pl.BlockSpec((3, 128, N), lambda i: (0, i, 0)),
            pl.BlockSpec((3, 1, 128), lambda i: (0, 0, i)),
            pl.BlockSpec((3, 1, N), lambda i: (0, 0, 0)),
        ],
        out_shape=[
            jax.ShapeDtypeStruct((N, N), jnp.float32),
            jax.ShapeDtypeStruct((3, N, N), jnp.bfloat16),
            jax.ShapeDtypeStruct((3, 1, N), jnp.float32),
            jax.ShapeDtypeStruct((3, 1, N), jnp.float32),
        ],
    )(A1, A2, A3)


def _k_hs3_body(x1_ref, x2_ref, x3_ref, w_ref, deg_ref, hsh_ref, hsl_ref):
    for v, x_ref in enumerate((x1_ref, x2_ref, x3_ref)):
        scale = lax.rsqrt(jnp.maximum(deg_ref[v, 0], 1.0))
        h = jnp.dot(x_ref[...], w_ref[v],
                    preferred_element_type=jnp.float32) * scale[:, None]
        hh, hl = _hilo(h)
        hsh_ref[v] = hh
        hsl_ref[v] = hl


def _scaled_proj3(x1, x2, x3, w, deg):
    f = w.shape[-1]
    mk = jax.ShapeDtypeStruct((3, N, f), jnp.bfloat16)
    return pl.pallas_call(
        _k_hs3_body,
        out_shape=[mk, mk],
    )(x1, x2, x3, w, deg)


def _k_agg_body(a_ref, hsh_ref, hsl_ref, cs_ref, b_ref, f_ref, *, act):
    a = a_ref[0]
    dn = (((0,), (0,)), ((), ()))
    agg = (lax.dot_general(a, hsh_ref[0], dn,
                           preferred_element_type=jnp.float32)
           + lax.dot_general(a, hsl_ref[0], dn,
                             preferred_element_type=jnp.float32))
    scale = lax.rsqrt(jnp.maximum(cs_ref[0, 0], 1.0))
    r = agg * scale[:, None] + b_ref[0, 0][None, :]
    f_ref[0] = jnp.maximum(r, 0.0) if act else r


def _agg3(A, hsh, hsl, cs, b, act):
    f = hsh.shape[-1]
    return pl.pallas_call(
        functools.partial(_k_agg_body, act=act),
        grid=(3,),
        in_specs=[
            pl.BlockSpec((1, N, N), lambda v: (v, 0, 0)),
            pl.BlockSpec((1, N, f), lambda v: (v, 0, 0)),
            pl.BlockSpec((1, N, f), lambda v: (v, 0, 0)),
            pl.BlockSpec((1, 1, N), lambda v: (v, 0, 0)),
            pl.BlockSpec((1, 1, f), lambda v: (v, 0, 0)),
        ],
        out_specs=pl.BlockSpec((1, N, f), lambda v: (v, 0, 0)),
        out_shape=jax.ShapeDtypeStruct((3, N, f), jnp.float32),
    )(A, hsh, hsl, cs, b)


def _k_fuse_body(f_ref, wf_ref, z_ref):
    acc = jnp.dot(f_ref[0], wf_ref[0], preferred_element_type=jnp.float32)
    acc += jnp.dot(f_ref[1], wf_ref[1], preferred_element_type=jnp.float32)
    acc += jnp.dot(f_ref[2], wf_ref[2], preferred_element_type=jnp.float32)
    m = jnp.max(acc, axis=1, keepdims=True)
    e = jnp.exp(acc - m)
    sm = e / jnp.sum(e, axis=1, keepdims=True)
    z_ref[...] = jnp.maximum(sm, 0.0)


def _fuse(fb, wf):
    return pl.pallas_call(
        _k_fuse_body,
        out_shape=jax.ShapeDtypeStruct((N, 64), jnp.float32),
    )(fb, wf)


def _k_gfn_body(adjin_ref, wg1_ref, bg1_ref, wg2_ref, bg2_ref, adjr_ref):
    h1 = jnp.dot(adjin_ref[...], wg1_ref[...],
                 preferred_element_type=jnp.float32) + bg1_ref[0][None, :]
    h1 = jnp.maximum(h1, 0.0)
    h2 = jnp.dot(h1, wg2_ref[...],
                 preferred_element_type=jnp.float32) + bg2_ref[0][None, :]
    adjr_ref[...] = jnp.round(jnp.clip(h2, 0.0, 1.0) + 0.1)


def _gfn(adjin, wg1, bg1, wg2, bg2):
    return pl.pallas_call(
        _k_gfn_body,
        grid=(8,),
        in_specs=[
            pl.BlockSpec((256, N), lambda i: (i, 0)),
            pl.BlockSpec((N, N // 2), lambda i: (0, 0)),
            pl.BlockSpec((1, N // 2), lambda i: (0, 0)),
            pl.BlockSpec((N // 2, N), lambda i: (0, 0)),
            pl.BlockSpec((1, N), lambda i: (0, 0)),
        ],
        out_specs=pl.BlockSpec((256, N), lambda i: (i, 0)),
        out_shape=jax.ShapeDtypeStruct((N, N), jnp.float32),
    )(adjin, wg1, bg1, wg2, bg2)


_CB = 512


def _k_abuild_body(ar_ref, art_ref, A_ref, s_ref):
    i = pl.program_id(0)
    j = pl.program_id(1)
    apn = ar_ref[...] + art_ref[...].T
    ri = lax.broadcasted_iota(jnp.int32, (_CB, _CB), 0) + i * _CB
    ci = lax.broadcasted_iota(jnp.int32, (_CB, _CB), 1) + j * _CB
    ablk = (apn != 0).astype(jnp.float32) + (ri == ci).astype(jnp.float32)
    A_ref[...] = ablk

    @pl.when(j == 0)
    def _():
        s_ref[...] = jnp.zeros_like(s_ref)

    s_ref[...] += jnp.sum(ablk, axis=1)[None, :]


def _build_consensus(adj_r):
    g = N // _CB
    return pl.pallas_call(
        _k_abuild_body,
        grid=(g, g),
        in_specs=[
            pl.BlockSpec((_CB, _CB), lambda i, j: (i, j)),
            pl.BlockSpec((_CB, _CB), lambda i, j: (j, i)),
        ],
        out_specs=[
            pl.BlockSpec((_CB, _CB), lambda i, j: (i, j)),
            pl.BlockSpec((1, _CB), lambda i, j: (0, i)),
        ],
        out_shape=[
            jax.ShapeDtypeStruct((N, N), jnp.float32),
            jax.ShapeDtypeStruct((1, N), jnp.float32),
        ],
    )(adj_r, adj_r)


def _k_proj_body(x_ref, w_ref, s_ref, u_ref):
    scale = lax.rsqrt(jnp.maximum(s_ref[0], 1.0))
    u_ref[...] = jnp.dot(x_ref[...], w_ref[...],
                         preferred_element_type=jnp.float32) * scale[:, None]


def _scaled_proj(x, w, s):
    f = w.shape[-1]
    return pl.pallas_call(
        _k_proj_body,
        out_shape=jax.ShapeDtypeStruct((N, f), jnp.float32),
    )(x, w, s)


def _k_dconv_body(a_ref, u_ref, s_ref, b_ref, o_ref, *, act):
    agg = jnp.dot(a_ref[...], u_ref[...], preferred_element_type=jnp.float32)
    scale = lax.rsqrt(jnp.maximum(s_ref[0], 1.0))
    r = agg * scale[:, None] + b_ref[0][None, :]
    o_ref[...] = jnp.maximum(r, 0.0) if act else r


def _dconv(A, u, s, b, act):
    f = u.shape[-1]
    return pl.pallas_call(
        functools.partial(_k_dconv_body, act=act),
        grid=(16,),
        in_specs=[
            pl.BlockSpec((128, N), lambda i: (i, 0)),
            pl.BlockSpec((N, f), lambda i: (0, 0)),
            pl.BlockSpec((1, 128), lambda i: (0, i)),
            pl.BlockSpec((1, f), lambda i: (0, 0)),
        ],
        out_specs=pl.BlockSpec((128, f), lambda i: (i, 0)),
        out_shape=jax.ShapeDtypeStruct((N, f), jnp.float32),
    )(A, u, s, b)


def _k_rec_body(fmb_ref, fm_ref, rec_ref):
    rec_ref[...] = lax.dot_general(fmb_ref[...], fm_ref[...],
                                   (((1,), (1,)), ((), ())),
                                   preferred_element_type=jnp.float32)


def _rec(fm):
    return pl.pallas_call(
        _k_rec_body,
        grid=(16,),
        in_specs=[
            pl.BlockSpec((128, 32), lambda i: (i, 0)),
            pl.BlockSpec((N, 32), lambda i: (0, 0)),
        ],
        out_specs=pl.BlockSpec((128, N), lambda i: (i, 0)),
        out_shape=jax.ShapeDtypeStruct((N, N), jnp.float32),
    )(fm, fm)


def kernel(data1, data2, data3, edge_index1, edge_index2, edge_index3,
           W1a, b1a, W1b, b1b, W2a, b2a, W2b, b2b, W3a, b3a, W3b, b3b,
           Wf1, Wf2, Wf3, Wm1, bm1, Wm2, bm2, Wg1, bg1, Wg2, bg2):
    A1 = _build_adjacency(edge_index1)
    A2 = _build_adjacency(edge_index2)
    A3 = _build_adjacency(edge_index3)

    adjin, Abf, rs, cs = _adjin_and_degrees(A1, A2, A3)

    wa = jnp.stack([W1a, W2a, W3a])
    ba = jnp.stack([b1a, b2a, b3a])[:, None, :]
    wb = jnp.stack([W1b, W2b, W3b])
    bb = jnp.stack([b1b, b2b, b3b])[:, None, :]
    wf = jnp.stack([Wf1, Wf2, Wf3])

    hsh_a, hsl_a = _scaled_proj3(data1, data2, data3, wa, rs)
    f_a = _agg3(Abf, hsh_a, hsl_a, cs, ba, act=True)
    hsh_b, hsl_b = _scaled_proj3(f_a[0], f_a[1], f_a[2], wb, rs)
    f_b = _agg3(Abf, hsh_b, hsl_b, cs, bb, act=False)

    z = _fuse(f_b, wf)

    adj_r = _gfn(adjin, Wg1, bg1[None, :], Wg2, bg2[None, :])
    Acons, s = _build_consensus(adj_r)

    u1 = _scaled_proj(z, Wm1, s)
    fm1 = _dconv(Acons, u1, s, bm1[None, :], act=True)
    u2 = _scaled_proj(fm1, Wm2, s)
    fm = _dconv(Acons, u2, s, bm2[None, :], act=False)

    rec = _rec(fm)
    return (adj_r, rec, rec, rec, fm)
